# Initial kernel scaffold; baseline (speedup 1.0000x reference)
import jax
import jax.numpy as jnp
from jax.experimental import pallas as pl


def kernel(x, edge_index, batch, W1, att_src1, att_dst1, b1, W2, att_src2, att_dst2, b2, w_ih0, w_hh0, b_ih0, b_hh0, w_ih1, w_hh1, b_ih1, b_hh1, W_lin, b_lin):
    raise NotImplementedError("write your pallas kernel here")


# speedup vs baseline: 23.0509x; 23.0509x over previous
import functools

import jax
import jax.numpy as jnp
from jax import lax
from jax.experimental import pallas as pl
from jax.experimental.pallas import tpu as pltpu
from jax.experimental.pallas import tpu_sc as plsc

NC = 2
NS = 16
LN = 16
NWK = NC * NS


def _leaky(v):
    return jnp.where(v < 0, v * jnp.float32(0.2), v)


def _elu(v):
    return jnp.where(v > 0, v, jnp.exp(jnp.minimum(v, 0.0)) - 1.0)


@functools.lru_cache(maxsize=None)
def _make_edge_pass(n, ch, e):
    W = 80
    CE = ch + LN
    ew = e // NWK
    assert e % NWK == 0 and ew % W == 0 and n % W == 0 and ch % LN == 0
    nwin = ew // W
    nz = n // W
    npt = (nz + NS - 1) // NS

    mesh = plsc.VectorSubcoreMesh(
        core_axis_name="c", subcore_axis_name="s",
        num_cores=NC, num_subcores=NS)

    def body(src_h, dst_h, tbl_h, asr_h, adr_h, m_h, acc_h,
             asr_v, adr_v, src_v, dst_v, g_v, rows_v, m_v, acc_sh, sem):
        c = lax.axis_index("c")
        s = lax.axis_index("s")
        wid = s * NC + c
        pltpu.sync_copy(asr_h, asr_v)
        pltpu.sync_copy(adr_h, adr_v)
        pltpu.sync_copy(m_h, m_v)

        zero16 = jnp.zeros((LN,), jnp.float32)
        lane0 = lax.iota(jnp.int32, LN) == 0

        def _zr(i, carry):
            r = i // (CE // LN)
            k = i % (CE // LN)
            rows_v[r, pl.ds(k * LN, LN)] = zero16
            return carry
        lax.fori_loop(0, W * CE // LN, _zr, 0)

        for kk in range(npt):
            chk = s * npt + kk

            @pl.when(chk < nz)
            def _():
                pltpu.sync_copy(rows_v, acc_sh.at[pl.ds(chk * W, W)])
        plsc.subcore_barrier()

        def win(i, carry):
            eb = wid * ew + i * W
            pltpu.sync_copy(src_h.at[pl.ds(eb, W)], src_v)
            pltpu.sync_copy(dst_h.at[pl.ds(eb, W)], dst_v)
            pltpu.async_copy(tbl_h.at[src_v], g_v, sem).wait()
            mv = m_v[...]
            for j in range(W // LN):
                si = src_v[pl.ds(j * LN, LN)]
                di = dst_v[pl.ds(j * LN, LN)]
                a = plsc.load_gather(asr_v, [si]) + plsc.load_gather(adr_v, [di])
                a = jnp.where(a < 0, a * jnp.float32(0.2), a)
                w = jnp.exp(a - mv)
                for rr in range(LN):
                    r = j * LN + rr
                    wr = w[rr]
                    for k in range(ch // LN):
                        rows_v[r, pl.ds(k * LN, LN)] = (
                            g_v[r, pl.ds(k * LN, LN)] * wr)
                    rows_v[r, pl.ds(ch, LN)] = jnp.where(lane0, wr, 0.0)
            pltpu.sync_copy(rows_v, acc_sh.at[dst_v], add=True)
            return carry
        lax.fori_loop(0, nwin, win, 0)
        plsc.subcore_barrier()

        for kk in range(npt):
            chk = s * npt + kk

            @pl.when(chk < nz)
            def _():
                pltpu.sync_copy(acc_sh.at[pl.ds(chk * W, W)],
                                acc_h.at[c, pl.ds(chk * W, W)])

    return pl.kernel(
        body,
        out_type=jax.ShapeDtypeStruct((NC, n, CE), jnp.float32),
        mesh=mesh,
        compiler_params=pltpu.CompilerParams(
            needs_layout_passes=False, use_tc_tiling_on_sc=False),
        scratch_types=[
            pltpu.VMEM((n,), jnp.float32),
            pltpu.VMEM((n,), jnp.float32),
            pltpu.VMEM((W,), jnp.int32),
            pltpu.VMEM((W,), jnp.int32),
            pltpu.VMEM((W, ch), jnp.float32),
            pltpu.VMEM((W, CE), jnp.float32),
            pltpu.VMEM((LN,), jnp.float32),
            pltpu.VMEM_SHARED((n, CE), jnp.float32),
            pltpu.SemaphoreType.DMA,
        ],
    )


def _tc_pre(x, W1, att8T, heads, ch):
    n, f_in = x.shape
    bn = 2000
    grid = n // bn
    k2 = 2 * heads

    def body(x_ref, w_ref, a_ref, *outs):
        i = pl.program_id(0)
        hrefs, asd_ref, st_ref = outs[:heads], outs[heads], outs[heads + 1]
        hb = jnp.dot(x_ref[...], w_ref[...], preferred_element_type=jnp.float32)
        for p in range(heads):
            hrefs[p][...] = hb[:, p * ch:(p + 1) * ch]
        a8 = jnp.dot(hb, a_ref[...], preferred_element_type=jnp.float32)
        asd_ref[...] = a8
        bm = jnp.broadcast_to(jnp.max(a8, axis=0)[None, :], (8, k2))
        st_ref[...] = jnp.where(i == 0, bm, jnp.maximum(st_ref[...], bm))

    return pl.pallas_call(
        body,
        grid=(grid,),
        in_specs=[pl.BlockSpec((bn, f_in), lambda i: (i, 0)),
                  pl.BlockSpec((f_in, heads * ch), lambda i: (0, 0)),
                  pl.BlockSpec((heads * ch, k2), lambda i: (0, 0))],
        out_specs=[pl.BlockSpec((bn, ch), lambda i: (i, 0))] * heads
        + [pl.BlockSpec((bn, k2), lambda i: (i, 0)),
           pl.BlockSpec((8, k2), lambda i: (0, 0))],
        out_shape=[jax.ShapeDtypeStruct((n, ch), jnp.float32)] * heads
        + [jax.ShapeDtypeStruct((n, k2), jnp.float32),
           jax.ShapeDtypeStruct((8, k2), jnp.float32)],
    )(x, W1, att8T)


def _tc_mid(accs, hs, asd1, st1, b1p, W2, att2T, heads, ch):
    n = hs[0].shape[0]
    bn = 2000
    grid = n // bn
    c2 = W2.shape[1]
    ce = ch + LN

    def body(*refs):
        it = iter(refs)
        acc_r = [next(it) for _ in range(heads)]
        h_r = [next(it) for _ in range(heads)]
        asd_r, st_r, b1_r, w2_r, a2_r = (next(it) for _ in range(5))
        h2_ref, asd2_ref, st2_ref = (next(it) for _ in range(3))
        i = pl.program_id(0)
        st = st_r[...]
        cat = []
        for p in range(heads):
            accsum = acc_r[p][0] + acc_r[p][1]
            acc = accsum[:, 0:ch]
            den = accsum[:, ch]
            asr = asd_r[:, p]
            adr = asd_r[:, heads + p]
            m = _leaky(st[0, p] + st[0, heads + p])
            wself = jnp.exp(_leaky(asr + adr) - m)
            num = acc + wself[:, None] * h_r[p][...]
            outp = num / (den + wself)[:, None] + b1_r[0, p * ch:(p + 1) * ch][None, :]
            cat.append(_elu(outp))
        hcat = jnp.concatenate(cat, axis=1)
        h2 = jnp.dot(hcat, w2_r[...], preferred_element_type=jnp.float32)
        h2_ref[...] = h2
        a2 = jnp.dot(h2, a2_r[...], preferred_element_type=jnp.float32)
        asd2_ref[...] = a2
        bm = jnp.broadcast_to(jnp.max(a2, axis=0)[None, :], (8, 8))
        st2_ref[...] = jnp.where(i == 0, bm, jnp.maximum(st2_ref[...], bm))

    full = lambda shape: pl.BlockSpec(shape, lambda i: tuple(0 for _ in shape))
    return pl.pallas_call(
        body,
        grid=(grid,),
        in_specs=(
            [pl.BlockSpec((NC, bn, ce), lambda i: (0, i, 0))] * heads
            + [pl.BlockSpec((bn, ch), lambda i: (i, 0))] * heads
            + [pl.BlockSpec((bn, 2 * heads), lambda i: (i, 0)),
               full((8, 2 * heads)), full((8, heads * ch)),
               full((heads * ch, c2)), full((c2, 8))]),
        out_specs=[pl.BlockSpec((bn, c2), lambda i: (i, 0)),
                   pl.BlockSpec((bn, 8), lambda i: (i, 0)),
                   full((8, 8))],
        out_shape=[jax.ShapeDtypeStruct((n, c2), jnp.float32),
                   jax.ShapeDtypeStruct((n, 8), jnp.float32),
                   jax.ShapeDtypeStruct((8, 8), jnp.float32)],
    )(*accs, *hs, asd1, st1, b1p, W2, att2T)


def _tc_fin(acc2, h2, asd2, st2, b2p, wih0T, bih0p, bhh0p,
            wih1T, bih1p, bhh1p, wlinT, blinp):
    n, ch = h2.shape
    bn = 2000
    grid = n // bn
    ce = ch + LN

    def body(a_r, h_r, asd_r, st_r, b2_r, wi0_r, bi0_r, bh0_r,
             wi1_r, bi1_r, bh1_r, wl_r, bl_r, out_ref):
        accsum = a_r[0] + a_r[1]
        acc = accsum[:, 0:ch]
        den = accsum[:, ch]
        st = st_r[...]
        m = _leaky(st[0, 0] + st[0, 1])
        wself = jnp.exp(_leaky(asd_r[:, 0] + asd_r[:, 1]) - m)
        num = acc + wself[:, None] * h_r[...]
        xg = _elu(num / (den + wself)[:, None] + b2_r[0, :][None, :])

        def gru0(xv, wi_r, bi_r, bh_r):
            gi = jnp.dot(xv, wi_r[...], preferred_element_type=jnp.float32)
            gi = gi + bi_r[0, :][None, :]
            bh = bh_r[0, :]
            r = jax.nn.sigmoid(gi[:, 0:ch] + bh[None, 0:ch])
            z = jax.nn.sigmoid(gi[:, ch:2 * ch] + bh[None, ch:2 * ch])
            nn = jnp.tanh(gi[:, 2 * ch:3 * ch] + r * bh[None, 2 * ch:3 * ch])
            return (1.0 - z) * nn

        h0 = gru0(xg, wi0_r, bi0_r, bh0_r)
        h1v = gru0(h0, wi1_r, bi1_r, bh1_r)
        out = jnp.dot(h1v, wl_r[...], preferred_element_type=jnp.float32)
        out_ref[...] = out + bl_r[0, 0]

    full = lambda shape: pl.BlockSpec(shape, lambda i: tuple(0 for _ in shape))
    return pl.pallas_call(
        body,
        grid=(grid,),
        in_specs=[pl.BlockSpec((NC, bn, ce), lambda i: (0, i, 0)),
                  pl.BlockSpec((bn, ch), lambda i: (i, 0)),
                  pl.BlockSpec((bn, 8), lambda i: (i, 0)),
                  full((8, 8)), full((8, ch)),
                  full((ch, 3 * ch)), full((8, 3 * ch)), full((8, 3 * ch)),
                  full((ch, 3 * ch)), full((8, 3 * ch)), full((8, 3 * ch)),
                  full((ch, 8)), full((8, 8))],
        out_specs=[pl.BlockSpec((bn, 8), lambda i: (i, 0))],
        out_shape=[jax.ShapeDtypeStruct((n, 8), jnp.float32)],
    )(acc2, h2, asd2, st2, b2p, wih0T, bih0p, bhh0p,
      wih1T, bih1p, bhh1p, wlinT, blinp)[0]


def kernel(x, edge_index, batch, W1, att_src1, att_dst1, b1, W2, att_src2,
           att_dst2, b2, w_ih0, w_hh0, b_ih0, b_hh0, w_ih1, w_hh1, b_ih1,
           b_hh1, W_lin, b_lin):
    n, f_in = x.shape
    heads, ch = att_src1.shape
    e = edge_index.shape[1]
    c2 = att_src2.shape[1]
    src = edge_index[0]
    dst = edge_index[1]

    cols = []
    for k in range(heads):
        cols.append(jnp.zeros((heads * ch,), jnp.float32)
                    .at[k * ch:(k + 1) * ch].set(att_src1[k]))
    for k in range(heads):
        cols.append(jnp.zeros((heads * ch,), jnp.float32)
                    .at[k * ch:(k + 1) * ch].set(att_dst1[k]))
    att8T = jnp.stack(cols, axis=1)

    att2T = jnp.zeros((c2, 8), jnp.float32)
    att2T = att2T.at[:, 0].set(att_src2[0]).at[:, 1].set(att_dst2[0])

    outs = _tc_pre(x, W1, att8T, heads, ch)
    hs, asd1, st1 = outs[:heads], outs[heads], outs[heads + 1]

    edge_pass = _make_edge_pass(n, ch, e)
    accs = []
    for p in range(heads):
        mp = _leaky(st1[0, p] + st1[0, heads + p])
        accs.append(edge_pass(
            src, dst, hs[p],
            asd1[:, p], asd1[:, heads + p],
            jnp.full((LN,), mp, jnp.float32)))

    b1p = jnp.broadcast_to(b1[None, :], (8, heads * ch))
    h2, asd2, st2 = _tc_mid(accs, hs, asd1, st1, b1p, W2, att2T, heads, ch)

    m2 = _leaky(st2[0, 0] + st2[0, 1])
    acc2 = edge_pass(
        src, dst, h2, asd2[:, 0], asd2[:, 1],
        jnp.full((LN,), m2, jnp.float32))

    pad8 = lambda v: jnp.broadcast_to(v[None, :], (8, v.shape[0]))
    out = _tc_fin(acc2, h2, asd2, st2, pad8(b2),
                  w_ih0.T, pad8(b_ih0), pad8(b_hh0),
                  w_ih1.T, pad8(b_ih1), pad8(b_hh1),
                  jnp.pad(W_lin.T, ((0, 0), (0, 7))),
                  jnp.broadcast_to(b_lin.reshape(1, 1), (8, 8)))
    return out[:, 0:1]


# speedup vs baseline: 26.5712x; 1.1527x over previous
import functools

import jax
import jax.numpy as jnp
from jax import lax
from jax.experimental import pallas as pl
from jax.experimental.pallas import tpu as pltpu
from jax.experimental.pallas import tpu_sc as plsc

NC = 2
NS = 16
LN = 16
NWK = NC * NS


def _leaky(v):
    return jnp.where(v < 0, v * jnp.float32(0.2), v)


def _elu(v):
    return jnp.where(v > 0, v, jnp.exp(jnp.minimum(v, 0.0)) - 1.0)


@functools.lru_cache(maxsize=None)
def _make_edge_pass(n, ch, e):
    W = 80
    CE = ch + LN
    ew = e // NWK
    assert e % NWK == 0 and ew % W == 0 and n % W == 0 and ch % LN == 0
    nwin = ew // W
    nz = n // W
    npt = (nz + NS - 1) // NS

    mesh = plsc.VectorSubcoreMesh(
        core_axis_name="c", subcore_axis_name="s",
        num_cores=NC, num_subcores=NS)

    def body(src_h, dst_h, tbl_h, asr_h, adr_h, m_h, acc_h,
             asr_v, adr_v, sidx, didx, g_v, rows_v, m_v, acc_sh,
             sem_i, sem_g):
        c = lax.axis_index("c")
        s = lax.axis_index("s")
        wid = s * NC + c
        pltpu.sync_copy(asr_h, asr_v)
        pltpu.sync_copy(adr_h, adr_v)
        pltpu.sync_copy(m_h, m_v)

        zero16 = jnp.zeros((LN,), jnp.float32)
        lane0 = lax.iota(jnp.int32, LN) == 0

        def _zr(i, carry):
            r = i // (CE // LN)
            k = i % (CE // LN)
            rows_v[r, pl.ds(k * LN, LN)] = zero16
            return carry
        lax.fori_loop(0, W * CE // LN, _zr, 0)

        for kk in range(npt):
            chk = s * npt + kk

            @pl.when(chk < nz)
            def _():
                pltpu.sync_copy(rows_v, acc_sh.at[pl.ds(chk * W, W)])
        plsc.subcore_barrier()

        base = wid * ew

        def start_idx(i):
            b = lax.rem(i, 4)
            pltpu.async_copy(src_h.at[pl.ds(base + i * W, W)],
                             sidx.at[b], sem_i)
            pltpu.async_copy(dst_h.at[pl.ds(base + i * W, W)],
                             didx.at[b], sem_i)

        def wait_idx(i):
            b = lax.rem(i, 4)
            pltpu.make_async_copy(src_h.at[pl.ds(base, W)],
                                  sidx.at[b], sem_i).wait()
            pltpu.make_async_copy(dst_h.at[pl.ds(base, W)],
                                  didx.at[b], sem_i).wait()

        def start_gather(i):
            b = lax.rem(i, 2)
            pltpu.async_copy(tbl_h.at[sidx.at[lax.rem(i, 4)]],
                             g_v.at[b], sem_g)

        def wait_gather(i):
            b = lax.rem(i, 2)
            pltpu.make_async_copy(tbl_h.at[sidx.at[lax.rem(i, 4)]],
                                  g_v.at[b], sem_g).wait()

        start_idx(jnp.int32(0))
        if nwin > 1:
            start_idx(jnp.int32(1))
        wait_idx(jnp.int32(0))
        start_gather(jnp.int32(0))

        def win(i, carry):
            @pl.when(i + 2 < nwin)
            def _():
                start_idx(i + 2)

            @pl.when(i + 1 < nwin)
            def _():
                wait_idx(i + 1)
                start_gather(i + 1)
            wait_gather(i)
            gb = lax.rem(i, 2)
            ib = lax.rem(i, 4)
            mv = m_v[...]
            for j in range(W // LN):
                si = sidx[ib, pl.ds(j * LN, LN)]
                di = didx[ib, pl.ds(j * LN, LN)]
                a = plsc.load_gather(asr_v, [si]) + plsc.load_gather(adr_v, [di])
                a = jnp.where(a < 0, a * jnp.float32(0.2), a)
                w = jnp.exp(a - mv)
                for rr in range(LN):
                    r = j * LN + rr
                    wr = w[rr]
                    for k in range(ch // LN):
                        rows_v[r, pl.ds(k * LN, LN)] = (
                            g_v[gb, r, pl.ds(k * LN, LN)] * wr)
                    rows_v[r, pl.ds(ch, LN)] = jnp.where(lane0, wr, 0.0)
            pltpu.sync_copy(rows_v, acc_sh.at[didx.at[ib]], add=True)
            return carry
        lax.fori_loop(0, nwin, win, 0)
        plsc.subcore_barrier()

        for kk in range(npt):
            chk = s * npt + kk

            @pl.when(chk < nz)
            def _():
                pltpu.sync_copy(acc_sh.at[pl.ds(chk * W, W)],
                                acc_h.at[c, pl.ds(chk * W, W)])

    return pl.kernel(
        body,
        out_type=jax.ShapeDtypeStruct((NC, n, CE), jnp.float32),
        mesh=mesh,
        compiler_params=pltpu.CompilerParams(
            needs_layout_passes=False, use_tc_tiling_on_sc=False),
        scratch_types=[
            pltpu.VMEM((n,), jnp.float32),
            pltpu.VMEM((n,), jnp.float32),
            pltpu.VMEM((4, W), jnp.int32),
            pltpu.VMEM((4, W), jnp.int32),
            pltpu.VMEM((2, W, ch), jnp.float32),
            pltpu.VMEM((W, CE), jnp.float32),
            pltpu.VMEM((LN,), jnp.float32),
            pltpu.VMEM_SHARED((n, CE), jnp.float32),
            pltpu.SemaphoreType.DMA,
            pltpu.SemaphoreType.DMA,
        ],
    )


def _tc_pre(x, W1, att8T, heads, ch):
    n, f_in = x.shape
    bn = 2000
    grid = n // bn
    k2 = 2 * heads

    def body(x_ref, w_ref, a_ref, *outs):
        i = pl.program_id(0)
        hrefs, asd_ref, st_ref = outs[:heads], outs[heads], outs[heads + 1]
        hb = jnp.dot(x_ref[...], w_ref[...], preferred_element_type=jnp.float32)
        for p in range(heads):
            hrefs[p][...] = hb[:, p * ch:(p + 1) * ch]
        a8 = jnp.dot(hb, a_ref[...], preferred_element_type=jnp.float32)
        asd_ref[...] = a8
        bm = jnp.broadcast_to(jnp.max(a8, axis=0)[None, :], (8, k2))
        st_ref[...] = jnp.where(i == 0, bm, jnp.maximum(st_ref[...], bm))

    return pl.pallas_call(
        body,
        grid=(grid,),
        in_specs=[pl.BlockSpec((bn, f_in), lambda i: (i, 0)),
                  pl.BlockSpec((f_in, heads * ch), lambda i: (0, 0)),
                  pl.BlockSpec((heads * ch, k2), lambda i: (0, 0))],
        out_specs=[pl.BlockSpec((bn, ch), lambda i: (i, 0))] * heads
        + [pl.BlockSpec((bn, k2), lambda i: (i, 0)),
           pl.BlockSpec((8, k2), lambda i: (0, 0))],
        out_shape=[jax.ShapeDtypeStruct((n, ch), jnp.float32)] * heads
        + [jax.ShapeDtypeStruct((n, k2), jnp.float32),
           jax.ShapeDtypeStruct((8, k2), jnp.float32)],
    )(x, W1, att8T)


def _tc_mid(accs, hs, asd1, st1, b1p, W2, att2T, heads, ch):
    n = hs[0].shape[0]
    bn = 2000
    grid = n // bn
    c2 = W2.shape[1]
    ce = ch + LN

    def body(*refs):
        it = iter(refs)
        acc_r = [next(it) for _ in range(heads)]
        h_r = [next(it) for _ in range(heads)]
        asd_r, st_r, b1_r, w2_r, a2_r = (next(it) for _ in range(5))
        h2_ref, asd2_ref, st2_ref = (next(it) for _ in range(3))
        i = pl.program_id(0)
        st = st_r[...]
        cat = []
        for p in range(heads):
            accsum = acc_r[p][0] + acc_r[p][1]
            acc = accsum[:, 0:ch]
            den = accsum[:, ch]
            asr = asd_r[:, p]
            adr = asd_r[:, heads + p]
            m = _leaky(st[0, p] + st[0, heads + p])
            wself = jnp.exp(_leaky(asr + adr) - m)
            num = acc + wself[:, None] * h_r[p][...]
            outp = num / (den + wself)[:, None] + b1_r[0, p * ch:(p + 1) * ch][None, :]
            cat.append(_elu(outp))
        hcat = jnp.concatenate(cat, axis=1)
        h2 = jnp.dot(hcat, w2_r[...], preferred_element_type=jnp.float32)
        h2_ref[...] = h2
        a2 = jnp.dot(h2, a2_r[...], preferred_element_type=jnp.float32)
        asd2_ref[...] = a2
        bm = jnp.broadcast_to(jnp.max(a2, axis=0)[None, :], (8, 8))
        st2_ref[...] = jnp.where(i == 0, bm, jnp.maximum(st2_ref[...], bm))

    full = lambda shape: pl.BlockSpec(shape, lambda i: tuple(0 for _ in shape))
    return pl.pallas_call(
        body,
        grid=(grid,),
        in_specs=(
            [pl.BlockSpec((NC, bn, ce), lambda i: (0, i, 0))] * heads
            + [pl.BlockSpec((bn, ch), lambda i: (i, 0))] * heads
            + [pl.BlockSpec((bn, 2 * heads), lambda i: (i, 0)),
               full((8, 2 * heads)), full((8, heads * ch)),
               full((heads * ch, c2)), full((c2, 8))]),
        out_specs=[pl.BlockSpec((bn, c2), lambda i: (i, 0)),
                   pl.BlockSpec((bn, 8), lambda i: (i, 0)),
                   full((8, 8))],
        out_shape=[jax.ShapeDtypeStruct((n, c2), jnp.float32),
                   jax.ShapeDtypeStruct((n, 8), jnp.float32),
                   jax.ShapeDtypeStruct((8, 8), jnp.float32)],
    )(*accs, *hs, asd1, st1, b1p, W2, att2T)


def _tc_fin(acc2, h2, asd2, st2, b2p, wih0T, bih0p, bhh0p,
            wih1T, bih1p, bhh1p, wlinT, blinp):
    n, ch = h2.shape
    bn = 2000
    grid = n // bn
    ce = ch + LN

    def body(a_r, h_r, asd_r, st_r, b2_r, wi0_r, bi0_r, bh0_r,
             wi1_r, bi1_r, bh1_r, wl_r, bl_r, out_ref):
        accsum = a_r[0] + a_r[1]
        acc = accsum[:, 0:ch]
        den = accsum[:, ch]
        st = st_r[...]
        m = _leaky(st[0, 0] + st[0, 1])
        wself = jnp.exp(_leaky(asd_r[:, 0] + asd_r[:, 1]) - m)
        num = acc + wself[:, None] * h_r[...]
        xg = _elu(num / (den + wself)[:, None] + b2_r[0, :][None, :])

        def gru0(xv, wi_r, bi_r, bh_r):
            gi = jnp.dot(xv, wi_r[...], preferred_element_type=jnp.float32)
            gi = gi + bi_r[0, :][None, :]
            bh = bh_r[0, :]
            r = jax.nn.sigmoid(gi[:, 0:ch] + bh[None, 0:ch])
            z = jax.nn.sigmoid(gi[:, ch:2 * ch] + bh[None, ch:2 * ch])
            nn = jnp.tanh(gi[:, 2 * ch:3 * ch] + r * bh[None, 2 * ch:3 * ch])
            return (1.0 - z) * nn

        h0 = gru0(xg, wi0_r, bi0_r, bh0_r)
        h1v = gru0(h0, wi1_r, bi1_r, bh1_r)
        out = jnp.dot(h1v, wl_r[...], preferred_element_type=jnp.float32)
        out_ref[...] = out + bl_r[0, 0]

    full = lambda shape: pl.BlockSpec(shape, lambda i: tuple(0 for _ in shape))
    return pl.pallas_call(
        body,
        grid=(grid,),
        in_specs=[pl.BlockSpec((NC, bn, ce), lambda i: (0, i, 0)),
                  pl.BlockSpec((bn, ch), lambda i: (i, 0)),
                  pl.BlockSpec((bn, 8), lambda i: (i, 0)),
                  full((8, 8)), full((8, ch)),
                  full((ch, 3 * ch)), full((8, 3 * ch)), full((8, 3 * ch)),
                  full((ch, 3 * ch)), full((8, 3 * ch)), full((8, 3 * ch)),
                  full((ch, 8)), full((8, 8))],
        out_specs=[pl.BlockSpec((bn, 8), lambda i: (i, 0))],
        out_shape=[jax.ShapeDtypeStruct((n, 8), jnp.float32)],
    )(acc2, h2, asd2, st2, b2p, wih0T, bih0p, bhh0p,
      wih1T, bih1p, bhh1p, wlinT, blinp)[0]


def kernel(x, edge_index, batch, W1, att_src1, att_dst1, b1, W2, att_src2,
           att_dst2, b2, w_ih0, w_hh0, b_ih0, b_hh0, w_ih1, w_hh1, b_ih1,
           b_hh1, W_lin, b_lin):
    n, f_in = x.shape
    heads, ch = att_src1.shape
    e = edge_index.shape[1]
    c2 = att_src2.shape[1]
    src = edge_index[0]
    dst = edge_index[1]

    cols = []
    for k in range(heads):
        cols.append(jnp.zeros((heads * ch,), jnp.float32)
                    .at[k * ch:(k + 1) * ch].set(att_src1[k]))
    for k in range(heads):
        cols.append(jnp.zeros((heads * ch,), jnp.float32)
                    .at[k * ch:(k + 1) * ch].set(att_dst1[k]))
    att8T = jnp.stack(cols, axis=1)

    att2T = jnp.zeros((c2, 8), jnp.float32)
    att2T = att2T.at[:, 0].set(att_src2[0]).at[:, 1].set(att_dst2[0])

    outs = _tc_pre(x, W1, att8T, heads, ch)
    hs, asd1, st1 = outs[:heads], outs[heads], outs[heads + 1]

    edge_pass = _make_edge_pass(n, ch, e)
    accs = []
    for p in range(heads):
        mp = _leaky(st1[0, p] + st1[0, heads + p])
        accs.append(edge_pass(
            src, dst, hs[p],
            asd1[:, p], asd1[:, heads + p],
            jnp.full((LN,), mp, jnp.float32)))

    b1p = jnp.broadcast_to(b1[None, :], (8, heads * ch))
    h2, asd2, st2 = _tc_mid(accs, hs, asd1, st1, b1p, W2, att2T, heads, ch)

    m2 = _leaky(st2[0, 0] + st2[0, 1])
    acc2 = edge_pass(
        src, dst, h2, asd2[:, 0], asd2[:, 1],
        jnp.full((LN,), m2, jnp.float32))

    pad8 = lambda v: jnp.broadcast_to(v[None, :], (8, v.shape[0]))
    out = _tc_fin(acc2, h2, asd2, st2, pad8(b2),
                  w_ih0.T, pad8(b_ih0), pad8(b_hh0),
                  w_ih1.T, pad8(b_ih1), pad8(b_hh1),
                  jnp.pad(W_lin.T, ((0, 0), (0, 7))),
                  jnp.broadcast_to(b_lin.reshape(1, 1), (8, 8)))
    return out[:, 0:1]


# speedup vs baseline: 30.5617x; 1.1502x over previous
import functools

import jax
import jax.numpy as jnp
from jax import lax
from jax.experimental import pallas as pl
from jax.experimental.pallas import tpu as pltpu
from jax.experimental.pallas import tpu_sc as plsc

NC = 2
NS = 16
LN = 16
NWK = NC * NS


def _leaky(v):
    return jnp.where(v < 0, v * jnp.float32(0.2), v)


def _elu(v):
    return jnp.where(v > 0, v, jnp.exp(jnp.minimum(v, 0.0)) - 1.0)


@functools.lru_cache(maxsize=None)
def _make_edge_pass(n, ch, e):
    W = 80
    CE = ch + LN
    ew = e // NWK
    assert e % NWK == 0 and ew % W == 0 and n % W == 0 and ch % LN == 0
    nwin = ew // W
    nz = n // W
    npt = (nz + NS - 1) // NS

    mesh = plsc.VectorSubcoreMesh(
        core_axis_name="c", subcore_axis_name="s",
        num_cores=NC, num_subcores=NS)

    def body(src_h, dst_h, tbl_h, asr_h, adr_h, m_h, acc_h,
             asr_v, adr_v, sidx, didx, g_v, rows_v, m_v, acc_sh,
             sem_i, sem_g, sem_s):
        c = lax.axis_index("c")
        s = lax.axis_index("s")
        wid = s * NC + c
        pltpu.sync_copy(asr_h, asr_v)
        pltpu.sync_copy(adr_h, adr_v)
        pltpu.sync_copy(m_h, m_v)

        zero16 = jnp.zeros((LN,), jnp.float32)
        lane0 = lax.iota(jnp.int32, LN) == 0

        def _zr(i, carry):
            b = i // (W * CE // LN)
            rk = i % (W * CE // LN)
            r = rk // (CE // LN)
            k = rk % (CE // LN)
            rows_v[b, r, pl.ds(k * LN, LN)] = zero16
            return carry
        lax.fori_loop(0, 2 * W * CE // LN, _zr, 0)

        for kk in range(npt):
            chk = s * npt + kk

            @pl.when(chk < nz)
            def _():
                pltpu.sync_copy(rows_v.at[0], acc_sh.at[pl.ds(chk * W, W)])
        plsc.subcore_barrier()

        base = wid * ew

        def start_idx(i):
            b = lax.rem(i, 4)
            pltpu.async_copy(src_h.at[pl.ds(base + i * W, W)],
                             sidx.at[b], sem_i)
            pltpu.async_copy(dst_h.at[pl.ds(base + i * W, W)],
                             didx.at[b], sem_i)

        def wait_idx(i):
            b = lax.rem(i, 4)
            pltpu.make_async_copy(src_h.at[pl.ds(base, W)],
                                  sidx.at[b], sem_i).wait()
            pltpu.make_async_copy(dst_h.at[pl.ds(base, W)],
                                  didx.at[b], sem_i).wait()

        def start_gather(i):
            b = lax.rem(i, 2)
            pltpu.async_copy(tbl_h.at[sidx.at[lax.rem(i, 4)]],
                             g_v.at[b], sem_g)

        def wait_gather(i):
            b = lax.rem(i, 2)
            pltpu.make_async_copy(tbl_h.at[sidx.at[lax.rem(i, 4)]],
                                  g_v.at[b], sem_g).wait()

        start_idx(jnp.int32(0))
        if nwin > 1:
            start_idx(jnp.int32(1))
        wait_idx(jnp.int32(0))
        start_gather(jnp.int32(0))

        def start_scatter(i):
            b = lax.rem(i, 2)
            pltpu.async_copy(rows_v.at[b], acc_sh.at[didx.at[lax.rem(i, 4)]],
                             sem_s, add=True)

        def wait_scatter(i):
            b = lax.rem(i, 2)
            pltpu.make_async_copy(rows_v.at[b],
                                  acc_sh.at[didx.at[lax.rem(i, 4)]],
                                  sem_s).wait()

        col_w = jnp.full((LN,), ch, jnp.int32)

        def win(i, carry):
            @pl.when(i >= 2)
            def _():
                wait_scatter(i - 2)

            @pl.when(i + 2 < nwin)
            def _():
                start_idx(i + 2)

            @pl.when(i + 1 < nwin)
            def _():
                wait_idx(i + 1)
                start_gather(i + 1)
            wait_gather(i)
            gb = lax.rem(i, 2)
            ib = lax.rem(i, 4)
            mv = m_v[...]
            for j in range(W // LN):
                si = sidx[ib, pl.ds(j * LN, LN)]
                di = didx[ib, pl.ds(j * LN, LN)]
                a = plsc.load_gather(asr_v, [si]) + plsc.load_gather(adr_v, [di])
                a = jnp.where(a < 0, a * jnp.float32(0.2), a)
                w = jnp.exp(a - mv)
                rowi = lax.iota(jnp.int32, LN) + j * LN
                plsc.store_scatter(rows_v, [jnp.full((LN,), gb, jnp.int32),
                                            rowi, col_w], w)
                for rr in range(LN):
                    r = j * LN + rr
                    wr = w[rr]
                    for k in range(ch // LN):
                        rows_v[gb, r, pl.ds(k * LN, LN)] = (
                            g_v[gb, r, pl.ds(k * LN, LN)] * wr)
            start_scatter(i)
            return carry
        lax.fori_loop(0, nwin, win, 0)
        if nwin >= 2:
            wait_scatter(jnp.int32(nwin - 2))
        wait_scatter(jnp.int32(nwin - 1))
        plsc.subcore_barrier()

        for kk in range(npt):
            chk = s * npt + kk

            @pl.when(chk < nz)
            def _():
                pltpu.sync_copy(acc_sh.at[pl.ds(chk * W, W)],
                                acc_h.at[c, pl.ds(chk * W, W)])

    return pl.kernel(
        body,
        out_type=jax.ShapeDtypeStruct((NC, n, CE), jnp.float32),
        mesh=mesh,
        compiler_params=pltpu.CompilerParams(
            needs_layout_passes=False, use_tc_tiling_on_sc=False),
        scratch_types=[
            pltpu.VMEM((n,), jnp.float32),
            pltpu.VMEM((n,), jnp.float32),
            pltpu.VMEM((4, W), jnp.int32),
            pltpu.VMEM((4, W), jnp.int32),
            pltpu.VMEM((2, W, ch), jnp.float32),
            pltpu.VMEM((2, W, CE), jnp.float32),
            pltpu.VMEM((LN,), jnp.float32),
            pltpu.VMEM_SHARED((n, CE), jnp.float32),
            pltpu.SemaphoreType.DMA,
            pltpu.SemaphoreType.DMA,
            pltpu.SemaphoreType.DMA,
        ],
    )


def _tc_pre(x, W1, att8T, heads, ch):
    n, f_in = x.shape
    bn = 2000
    grid = n // bn
    k2 = 2 * heads

    def body(x_ref, w_ref, a_ref, *outs):
        i = pl.program_id(0)
        hrefs, asd_ref, st_ref = outs[:heads], outs[heads], outs[heads + 1]
        hb = jnp.dot(x_ref[...], w_ref[...], preferred_element_type=jnp.float32)
        for p in range(heads):
            hrefs[p][...] = hb[:, p * ch:(p + 1) * ch]
        a8 = jnp.dot(hb, a_ref[...], preferred_element_type=jnp.float32)
        asd_ref[...] = a8
        bm = jnp.broadcast_to(jnp.max(a8, axis=0)[None, :], (8, k2))
        st_ref[...] = jnp.where(i == 0, bm, jnp.maximum(st_ref[...], bm))

    return pl.pallas_call(
        body,
        grid=(grid,),
        in_specs=[pl.BlockSpec((bn, f_in), lambda i: (i, 0)),
                  pl.BlockSpec((f_in, heads * ch), lambda i: (0, 0)),
                  pl.BlockSpec((heads * ch, k2), lambda i: (0, 0))],
        out_specs=[pl.BlockSpec((bn, ch), lambda i: (i, 0))] * heads
        + [pl.BlockSpec((bn, k2), lambda i: (i, 0)),
           pl.BlockSpec((8, k2), lambda i: (0, 0))],
        out_shape=[jax.ShapeDtypeStruct((n, ch), jnp.float32)] * heads
        + [jax.ShapeDtypeStruct((n, k2), jnp.float32),
           jax.ShapeDtypeStruct((8, k2), jnp.float32)],
    )(x, W1, att8T)


def _tc_mid(accs, hs, asd1, st1, b1p, W2, att2T, heads, ch):
    n = hs[0].shape[0]
    bn = 2000
    grid = n // bn
    c2 = W2.shape[1]
    ce = ch + LN

    def body(*refs):
        it = iter(refs)
        acc_r = [next(it) for _ in range(heads)]
        h_r = [next(it) for _ in range(heads)]
        asd_r, st_r, b1_r, w2_r, a2_r = (next(it) for _ in range(5))
        h2_ref, asd2_ref, st2_ref = (next(it) for _ in range(3))
        i = pl.program_id(0)
        st = st_r[...]
        cat = []
        for p in range(heads):
            accsum = acc_r[p][0] + acc_r[p][1]
            acc = accsum[:, 0:ch]
            den = accsum[:, ch]
            asr = asd_r[:, p]
            adr = asd_r[:, heads + p]
            m = _leaky(st[0, p] + st[0, heads + p])
            wself = jnp.exp(_leaky(asr + adr) - m)
            num = acc + wself[:, None] * h_r[p][...]
            outp = num / (den + wself)[:, None] + b1_r[0, p * ch:(p + 1) * ch][None, :]
            cat.append(_elu(outp))
        hcat = jnp.concatenate(cat, axis=1)
        h2 = jnp.dot(hcat, w2_r[...], preferred_element_type=jnp.float32)
        h2_ref[...] = h2
        a2 = jnp.dot(h2, a2_r[...], preferred_element_type=jnp.float32)
        asd2_ref[...] = a2
        bm = jnp.broadcast_to(jnp.max(a2, axis=0)[None, :], (8, 8))
        st2_ref[...] = jnp.where(i == 0, bm, jnp.maximum(st2_ref[...], bm))

    full = lambda shape: pl.BlockSpec(shape, lambda i: tuple(0 for _ in shape))
    return pl.pallas_call(
        body,
        grid=(grid,),
        in_specs=(
            [pl.BlockSpec((NC, bn, ce), lambda i: (0, i, 0))] * heads
            + [pl.BlockSpec((bn, ch), lambda i: (i, 0))] * heads
            + [pl.BlockSpec((bn, 2 * heads), lambda i: (i, 0)),
               full((8, 2 * heads)), full((8, heads * ch)),
               full((heads * ch, c2)), full((c2, 8))]),
        out_specs=[pl.BlockSpec((bn, c2), lambda i: (i, 0)),
                   pl.BlockSpec((bn, 8), lambda i: (i, 0)),
                   full((8, 8))],
        out_shape=[jax.ShapeDtypeStruct((n, c2), jnp.float32),
                   jax.ShapeDtypeStruct((n, 8), jnp.float32),
                   jax.ShapeDtypeStruct((8, 8), jnp.float32)],
    )(*accs, *hs, asd1, st1, b1p, W2, att2T)


def _tc_fin(acc2, h2, asd2, st2, b2p, wih0T, bih0p, bhh0p,
            wih1T, bih1p, bhh1p, wlinT, blinp):
    n, ch = h2.shape
    bn = 2000
    grid = n // bn
    ce = ch + LN

    def body(a_r, h_r, asd_r, st_r, b2_r, wi0_r, bi0_r, bh0_r,
             wi1_r, bi1_r, bh1_r, wl_r, bl_r, out_ref):
        accsum = a_r[0] + a_r[1]
        acc = accsum[:, 0:ch]
        den = accsum[:, ch]
        st = st_r[...]
        m = _leaky(st[0, 0] + st[0, 1])
        wself = jnp.exp(_leaky(asd_r[:, 0] + asd_r[:, 1]) - m)
        num = acc + wself[:, None] * h_r[...]
        xg = _elu(num / (den + wself)[:, None] + b2_r[0, :][None, :])

        def gru0(xv, wi_r, bi_r, bh_r):
            gi = jnp.dot(xv, wi_r[...], preferred_element_type=jnp.float32)
            gi = gi + bi_r[0, :][None, :]
            bh = bh_r[0, :]
            r = jax.nn.sigmoid(gi[:, 0:ch] + bh[None, 0:ch])
            z = jax.nn.sigmoid(gi[:, ch:2 * ch] + bh[None, ch:2 * ch])
            nn = jnp.tanh(gi[:, 2 * ch:3 * ch] + r * bh[None, 2 * ch:3 * ch])
            return (1.0 - z) * nn

        h0 = gru0(xg, wi0_r, bi0_r, bh0_r)
        h1v = gru0(h0, wi1_r, bi1_r, bh1_r)
        out = jnp.dot(h1v, wl_r[...], preferred_element_type=jnp.float32)
        out_ref[...] = out + bl_r[0, 0]

    full = lambda shape: pl.BlockSpec(shape, lambda i: tuple(0 for _ in shape))
    return pl.pallas_call(
        body,
        grid=(grid,),
        in_specs=[pl.BlockSpec((NC, bn, ce), lambda i: (0, i, 0)),
                  pl.BlockSpec((bn, ch), lambda i: (i, 0)),
                  pl.BlockSpec((bn, 8), lambda i: (i, 0)),
                  full((8, 8)), full((8, ch)),
                  full((ch, 3 * ch)), full((8, 3 * ch)), full((8, 3 * ch)),
                  full((ch, 3 * ch)), full((8, 3 * ch)), full((8, 3 * ch)),
                  full((ch, 8)), full((8, 8))],
        out_specs=[pl.BlockSpec((bn, 8), lambda i: (i, 0))],
        out_shape=[jax.ShapeDtypeStruct((n, 8), jnp.float32)],
    )(acc2, h2, asd2, st2, b2p, wih0T, bih0p, bhh0p,
      wih1T, bih1p, bhh1p, wlinT, blinp)[0]


def kernel(x, edge_index, batch, W1, att_src1, att_dst1, b1, W2, att_src2,
           att_dst2, b2, w_ih0, w_hh0, b_ih0, b_hh0, w_ih1, w_hh1, b_ih1,
           b_hh1, W_lin, b_lin):
    n, f_in = x.shape
    heads, ch = att_src1.shape
    e = edge_index.shape[1]
    c2 = att_src2.shape[1]
    src = edge_index[0]
    dst = edge_index[1]

    cols = []
    for k in range(heads):
        cols.append(jnp.zeros((heads * ch,), jnp.float32)
                    .at[k * ch:(k + 1) * ch].set(att_src1[k]))
    for k in range(heads):
        cols.append(jnp.zeros((heads * ch,), jnp.float32)
                    .at[k * ch:(k + 1) * ch].set(att_dst1[k]))
    att8T = jnp.stack(cols, axis=1)

    att2T = jnp.zeros((c2, 8), jnp.float32)
    att2T = att2T.at[:, 0].set(att_src2[0]).at[:, 1].set(att_dst2[0])

    outs = _tc_pre(x, W1, att8T, heads, ch)
    hs, asd1, st1 = outs[:heads], outs[heads], outs[heads + 1]

    edge_pass = _make_edge_pass(n, ch, e)
    accs = []
    for p in range(heads):
        mp = _leaky(st1[0, p] + st1[0, heads + p])
        accs.append(edge_pass(
            src, dst, hs[p],
            asd1[:, p], asd1[:, heads + p],
            jnp.full((LN,), mp, jnp.float32)))

    b1p = jnp.broadcast_to(b1[None, :], (8, heads * ch))
    h2, asd2, st2 = _tc_mid(accs, hs, asd1, st1, b1p, W2, att2T, heads, ch)

    m2 = _leaky(st2[0, 0] + st2[0, 1])
    acc2 = edge_pass(
        src, dst, h2, asd2[:, 0], asd2[:, 1],
        jnp.full((LN,), m2, jnp.float32))

    pad8 = lambda v: jnp.broadcast_to(v[None, :], (8, v.shape[0]))
    out = _tc_fin(acc2, h2, asd2, st2, pad8(b2),
                  w_ih0.T, pad8(b_ih0), pad8(b_hh0),
                  w_ih1.T, pad8(b_ih1), pad8(b_hh1),
                  jnp.pad(W_lin.T, ((0, 0), (0, 7))),
                  jnp.broadcast_to(b_lin.reshape(1, 1), (8, 8)))
    return out[:, 0:1]


# speedup vs baseline: 30.6719x; 1.0036x over previous
import functools

import jax
import jax.numpy as jnp
from jax import lax
from jax.experimental import pallas as pl
from jax.experimental.pallas import tpu as pltpu
from jax.experimental.pallas import tpu_sc as plsc

NC = 2
NS = 16
LN = 16
NWK = NC * NS


def _leaky(v):
    return jnp.where(v < 0, v * jnp.float32(0.2), v)


def _elu(v):
    return jnp.where(v > 0, v, jnp.exp(jnp.minimum(v, 0.0)) - 1.0)


@functools.lru_cache(maxsize=None)
def _make_edge_pass(n, ch, e):
    W = 80
    CE = ch + LN
    ew = e // NWK
    assert e % NWK == 0 and ew % W == 0 and n % W == 0 and ch % LN == 0
    nwin = ew // W
    nz = n // W
    npt = (nz + NS - 1) // NS

    mesh = plsc.VectorSubcoreMesh(
        core_axis_name="c", subcore_axis_name="s",
        num_cores=NC, num_subcores=NS)

    def body(src_h, dst_h, tbl_h, asr_h, adr_h, m_h, acc_h,
             asr_v, adr_v, sidx, didx, g_v, rows_v, m_v, acc_sh,
             sem_i, sem_g, sem_s):
        c = lax.axis_index("c")
        s = lax.axis_index("s")
        wid = s * NC + c
        pltpu.sync_copy(asr_h, asr_v)
        pltpu.sync_copy(adr_h, adr_v)
        pltpu.sync_copy(m_h, m_v)

        zero16 = jnp.zeros((LN,), jnp.float32)

        def _zr(i, carry):
            b = i // (W * CE // LN)
            rk = i % (W * CE // LN)
            r = rk // (CE // LN)
            k = rk % (CE // LN)
            rows_v[b, r, pl.ds(k * LN, LN)] = zero16
            return carry
        lax.fori_loop(0, 2 * W * CE // LN, _zr, 0)

        for kk in range(npt):
            chk = s * npt + kk

            @pl.when(chk < nz)
            def _():
                pltpu.sync_copy(rows_v.at[0], acc_sh.at[pl.ds(chk * W, W)])
        plsc.subcore_barrier()

        base = wid * ew

        def start_idx(i):
            b = lax.rem(i, 4)
            pltpu.async_copy(src_h.at[pl.ds(base + i * W, W)],
                             sidx.at[b], sem_i)
            pltpu.async_copy(dst_h.at[pl.ds(base + i * W, W)],
                             didx.at[b], sem_i)

        def wait_idx(i):
            b = lax.rem(i, 4)
            pltpu.make_async_copy(src_h.at[pl.ds(base, W)],
                                  sidx.at[b], sem_i).wait()
            pltpu.make_async_copy(dst_h.at[pl.ds(base, W)],
                                  didx.at[b], sem_i).wait()

        def start_gather(i):
            b = lax.rem(i, 2)
            pltpu.async_copy(tbl_h.at[sidx.at[lax.rem(i, 4)]],
                             g_v.at[b], sem_g)

        def wait_gather(i):
            b = lax.rem(i, 2)
            pltpu.make_async_copy(tbl_h.at[sidx.at[lax.rem(i, 4)]],
                                  g_v.at[b], sem_g).wait()

        start_idx(jnp.int32(0))
        if nwin > 1:
            start_idx(jnp.int32(1))
        wait_idx(jnp.int32(0))
        start_gather(jnp.int32(0))

        def start_scatter(i):
            b = lax.rem(i, 2)
            pltpu.async_copy(rows_v.at[b], acc_sh.at[didx.at[lax.rem(i, 4)]],
                             sem_s, add=True)

        def wait_scatter(i):
            b = lax.rem(i, 2)
            pltpu.make_async_copy(rows_v.at[b],
                                  acc_sh.at[didx.at[lax.rem(i, 4)]],
                                  sem_s).wait()

        col_w = jnp.full((LN,), ch, jnp.int32)

        def win(i, carry):
            @pl.when(i >= 2)
            def _():
                wait_scatter(i - 2)

            @pl.when(i + 2 < nwin)
            def _():
                start_idx(i + 2)

            @pl.when(i + 1 < nwin)
            def _():
                wait_idx(i + 1)
                start_gather(i + 1)
            wait_gather(i)
            gb = lax.rem(i, 2)
            ib = lax.rem(i, 4)
            mv = m_v[...]
            for j in range(W // LN):
                si = sidx[ib, pl.ds(j * LN, LN)]
                di = didx[ib, pl.ds(j * LN, LN)]
                a = plsc.load_gather(asr_v, [si]) + plsc.load_gather(adr_v, [di])
                a = jnp.where(a < 0, a * jnp.float32(0.2), a)
                w = jnp.exp(a - mv)
                rowi = lax.iota(jnp.int32, LN) + j * LN
                plsc.store_scatter(rows_v, [jnp.full((LN,), gb, jnp.int32),
                                            rowi, col_w], w)
                for rr in range(LN):
                    r = j * LN + rr
                    wr = w[rr]
                    for k in range(ch // LN):
                        rows_v[gb, r, pl.ds(k * LN, LN)] = (
                            g_v[gb, r, pl.ds(k * LN, LN)] * wr)
            start_scatter(i)
            return carry
        lax.fori_loop(0, nwin, win, 0)
        if nwin >= 2:
            wait_scatter(jnp.int32(nwin - 2))
        wait_scatter(jnp.int32(nwin - 1))
        plsc.subcore_barrier()

        for kk in range(npt):
            chk = s * npt + kk

            @pl.when(chk < nz)
            def _():
                pltpu.sync_copy(acc_sh.at[pl.ds(chk * W, W)],
                                acc_h.at[c, pl.ds(chk * W, W)])

    return pl.kernel(
        body,
        out_type=jax.ShapeDtypeStruct((NC, n, CE), jnp.float32),
        mesh=mesh,
        compiler_params=pltpu.CompilerParams(
            needs_layout_passes=False, use_tc_tiling_on_sc=False),
        scratch_types=[
            pltpu.VMEM((n,), jnp.float32),
            pltpu.VMEM((n,), jnp.float32),
            pltpu.VMEM((4, W), jnp.int32),
            pltpu.VMEM((4, W), jnp.int32),
            pltpu.VMEM((2, W, ch), jnp.float32),
            pltpu.VMEM((2, W, CE), jnp.float32),
            pltpu.VMEM((LN,), jnp.float32),
            pltpu.VMEM_SHARED((n, CE), jnp.float32),
            pltpu.SemaphoreType.DMA,
            pltpu.SemaphoreType.DMA,
            pltpu.SemaphoreType.DMA,
        ],
    )


def _tc_pre(x, W1, att8T, heads, ch):
    n, f_in = x.shape
    bn = 2000
    grid = n // bn
    k2 = 2 * heads

    def body(x_ref, w_ref, a_ref, *outs):
        i = pl.program_id(0)
        hrefs, asd_ref, st_ref = outs[:heads], outs[heads], outs[heads + 1]
        hb = jnp.dot(x_ref[...], w_ref[...], preferred_element_type=jnp.float32)
        for p in range(heads):
            hrefs[p][...] = hb[:, p * ch:(p + 1) * ch]
        a8 = jnp.dot(hb, a_ref[...], preferred_element_type=jnp.float32)
        asd_ref[...] = a8
        bm = jnp.broadcast_to(jnp.max(a8, axis=0)[None, :], (8, k2))
        st_ref[...] = jnp.where(i == 0, bm, jnp.maximum(st_ref[...], bm))

    return pl.pallas_call(
        body,
        grid=(grid,),
        in_specs=[pl.BlockSpec((bn, f_in), lambda i: (i, 0)),
                  pl.BlockSpec((f_in, heads * ch), lambda i: (0, 0)),
                  pl.BlockSpec((heads * ch, k2), lambda i: (0, 0))],
        out_specs=[pl.BlockSpec((bn, ch), lambda i: (i, 0))] * heads
        + [pl.BlockSpec((bn, k2), lambda i: (i, 0)),
           pl.BlockSpec((8, k2), lambda i: (0, 0))],
        out_shape=[jax.ShapeDtypeStruct((n, ch), jnp.float32)] * heads
        + [jax.ShapeDtypeStruct((n, k2), jnp.float32),
           jax.ShapeDtypeStruct((8, k2), jnp.float32)],
    )(x, W1, att8T)


def _tc_mid(accs, hs, asd1, st1, b1p, W2, att2T, heads, ch):
    n = hs[0].shape[0]
    bn = 2000
    grid = n // bn
    c2 = W2.shape[1]
    ce = ch + LN

    def body(*refs):
        it = iter(refs)
        acc_r = [next(it) for _ in range(heads)]
        h_r = [next(it) for _ in range(heads)]
        asd_r, st_r, b1_r, w2_r, a2_r = (next(it) for _ in range(5))
        h2_ref, asd2_ref, st2_ref = (next(it) for _ in range(3))
        i = pl.program_id(0)
        st = st_r[...]
        cat = []
        for p in range(heads):
            accsum = acc_r[p][0] + acc_r[p][1]
            acc = accsum[:, 0:ch]
            den = accsum[:, ch]
            asr = asd_r[:, p]
            adr = asd_r[:, heads + p]
            m = _leaky(st[0, p] + st[0, heads + p])
            wself = jnp.exp(_leaky(asr + adr) - m)
            num = acc + wself[:, None] * h_r[p][...]
            outp = num / (den + wself)[:, None] + b1_r[0, p * ch:(p + 1) * ch][None, :]
            cat.append(_elu(outp))
        hcat = jnp.concatenate(cat, axis=1)
        h2 = jnp.dot(hcat, w2_r[...], preferred_element_type=jnp.float32)
        h2_ref[...] = h2
        a2 = jnp.dot(h2, a2_r[...], preferred_element_type=jnp.float32)
        asd2_ref[...] = a2
        bm = jnp.broadcast_to(jnp.max(a2, axis=0)[None, :], (8, 8))
        st2_ref[...] = jnp.where(i == 0, bm, jnp.maximum(st2_ref[...], bm))

    full = lambda shape: pl.BlockSpec(shape, lambda i: tuple(0 for _ in shape))
    return pl.pallas_call(
        body,
        grid=(grid,),
        in_specs=(
            [pl.BlockSpec((NC, bn, ce), lambda i: (0, i, 0))] * heads
            + [pl.BlockSpec((bn, ch), lambda i: (i, 0))] * heads
            + [pl.BlockSpec((bn, 2 * heads), lambda i: (i, 0)),
               full((8, 2 * heads)), full((8, heads * ch)),
               full((heads * ch, c2)), full((c2, 8))]),
        out_specs=[pl.BlockSpec((bn, c2), lambda i: (i, 0)),
                   pl.BlockSpec((bn, 8), lambda i: (i, 0)),
                   full((8, 8))],
        out_shape=[jax.ShapeDtypeStruct((n, c2), jnp.float32),
                   jax.ShapeDtypeStruct((n, 8), jnp.float32),
                   jax.ShapeDtypeStruct((8, 8), jnp.float32)],
    )(*accs, *hs, asd1, st1, b1p, W2, att2T)


def _tc_fin(acc2, h2, asd2, st2, b2p, wih0T, bih0p, bhh0p,
            wih1T, bih1p, bhh1p, wlinT, blinp):
    n, ch = h2.shape
    bn = 2000
    grid = n // bn
    ce = ch + LN

    def body(a_r, h_r, asd_r, st_r, b2_r, wi0_r, bi0_r, bh0_r,
             wi1_r, bi1_r, bh1_r, wl_r, bl_r, out_ref):
        accsum = a_r[0] + a_r[1]
        acc = accsum[:, 0:ch]
        den = accsum[:, ch]
        st = st_r[...]
        m = _leaky(st[0, 0] + st[0, 1])
        wself = jnp.exp(_leaky(asd_r[:, 0] + asd_r[:, 1]) - m)
        num = acc + wself[:, None] * h_r[...]
        xg = _elu(num / (den + wself)[:, None] + b2_r[0, :][None, :])

        def gru0(xv, wi_r, bi_r, bh_r):
            gi = jnp.dot(xv, wi_r[...], preferred_element_type=jnp.float32)
            gi = gi + bi_r[0, :][None, :]
            bh = bh_r[0, :]
            r = jax.nn.sigmoid(gi[:, 0:ch] + bh[None, 0:ch])
            z = jax.nn.sigmoid(gi[:, ch:2 * ch] + bh[None, ch:2 * ch])
            nn = jnp.tanh(gi[:, 2 * ch:3 * ch] + r * bh[None, 2 * ch:3 * ch])
            return (1.0 - z) * nn

        h0 = gru0(xg, wi0_r, bi0_r, bh0_r)
        h1v = gru0(h0, wi1_r, bi1_r, bh1_r)
        out = jnp.dot(h1v, wl_r[...], preferred_element_type=jnp.float32)
        out_ref[...] = out + bl_r[0, 0]

    full = lambda shape: pl.BlockSpec(shape, lambda i: tuple(0 for _ in shape))
    return pl.pallas_call(
        body,
        grid=(grid,),
        in_specs=[pl.BlockSpec((NC, bn, ce), lambda i: (0, i, 0)),
                  pl.BlockSpec((bn, ch), lambda i: (i, 0)),
                  pl.BlockSpec((bn, 8), lambda i: (i, 0)),
                  full((8, 8)), full((8, ch)),
                  full((ch, 3 * ch)), full((8, 3 * ch)), full((8, 3 * ch)),
                  full((ch, 3 * ch)), full((8, 3 * ch)), full((8, 3 * ch)),
                  full((ch, 8)), full((8, 8))],
        out_specs=[pl.BlockSpec((bn, 8), lambda i: (i, 0))],
        out_shape=[jax.ShapeDtypeStruct((n, 8), jnp.float32)],
    )(acc2, h2, asd2, st2, b2p, wih0T, bih0p, bhh0p,
      wih1T, bih1p, bhh1p, wlinT, blinp)[0]


def kernel(x, edge_index, batch, W1, att_src1, att_dst1, b1, W2, att_src2,
           att_dst2, b2, w_ih0, w_hh0, b_ih0, b_hh0, w_ih1, w_hh1, b_ih1,
           b_hh1, W_lin, b_lin):
    n, f_in = x.shape
    heads, ch = att_src1.shape
    e = edge_index.shape[1]
    c2 = att_src2.shape[1]
    src = edge_index[0]
    dst = edge_index[1]

    cols = []
    for k in range(heads):
        cols.append(jnp.zeros((heads * ch,), jnp.float32)
                    .at[k * ch:(k + 1) * ch].set(att_src1[k]))
    for k in range(heads):
        cols.append(jnp.zeros((heads * ch,), jnp.float32)
                    .at[k * ch:(k + 1) * ch].set(att_dst1[k]))
    att8T = jnp.stack(cols, axis=1)

    att2T = jnp.zeros((c2, 8), jnp.float32)
    att2T = att2T.at[:, 0].set(att_src2[0]).at[:, 1].set(att_dst2[0])

    outs = _tc_pre(x, W1, att8T, heads, ch)
    hs, asd1, st1 = outs[:heads], outs[heads], outs[heads + 1]

    edge_pass = _make_edge_pass(n, ch, e)
    accs = []
    for p in range(heads):
        mp = _leaky(st1[0, p] + st1[0, heads + p])
        accs.append(edge_pass(
            src, dst, hs[p],
            asd1[:, p], asd1[:, heads + p],
            jnp.full((LN,), mp, jnp.float32)))

    b1p = jnp.broadcast_to(b1[None, :], (8, heads * ch))
    h2, asd2, st2 = _tc_mid(accs, hs, asd1, st1, b1p, W2, att2T, heads, ch)

    m2 = _leaky(st2[0, 0] + st2[0, 1])
    acc2 = edge_pass(
        src, dst, h2, asd2[:, 0], asd2[:, 1],
        jnp.full((LN,), m2, jnp.float32))

    pad8 = lambda v: jnp.broadcast_to(v[None, :], (8, v.shape[0]))
    out = _tc_fin(acc2, h2, asd2, st2, pad8(b2),
                  w_ih0.T, pad8(b_ih0), pad8(b_hh0),
                  w_ih1.T, pad8(b_ih1), pad8(b_hh1),
                  jnp.pad(W_lin.T, ((0, 0), (0, 7))),
                  jnp.broadcast_to(b_lin.reshape(1, 1), (8, 8)))
    return out[:, 0:1]
